# baseline (speedup 1.0000x reference)
import jax
import jax.numpy as jnp
from jax import lax
from jax.experimental import pallas as pl
from jax.experimental.pallas import tpu as pltpu
from jax.experimental.pallas import tpu_sc as plsc

SEQ = 200
BATCH = 1024
DM = 64
NV = 4
LANES = 16
NC, NS = 2, 16
NW = NC * NS
ROWS_PER_W = BATCH // NW
CH = 40
NCHUNK = SEQ // CH
WTOK = ROWS_PER_W * SEQ
IPAD = 208
TOK = BATCH * SEQ
IDXB = NV * CH
HG = IDXB // 2


def _body(xTc, Wd, WL4, WR5, W6f, out,
          xidxA, xidxB, pidx_s, pidx_f, w6v, wrc, gbA, gbB, gb45,
          semA, semB, isemA, isemB, wsem):
    wid = lax.axis_index("s") * NC + lax.axis_index("c")
    tbase = wid * WTOK
    gbase = wid * ROWS_PER_W * NCHUNK

    iota = lax.iota(jnp.int32, LANES)
    for i in range(IPAD // LANES):
        s = iota + (i * LANES)
        pidx_s[pl.ds(i * LANES, LANES)] = jnp.minimum(s, SEQ - 1)
        pidx_f[pl.ds(i * LANES, LANES)] = jnp.clip(s - 149, 0, 50)

    pltpu.sync_copy(W6f, w6v)
    w6lo = [w6v[pl.ds(c * LANES, LANES)] for c in range(DM // LANES)]
    w6hi = [w6v[pl.ds(DM + c * LANES, LANES)] for c in range(DM // LANES)]
    for c in range(NCHUNK):
        dst = gb45.at[c]
        pltpu.async_copy(WL4.at[pidx_s.at[pl.ds(c * CH, CH)]],
                         dst, semA).wait()
        pltpu.async_copy(WR5.at[pidx_f.at[pl.ds(c * CH, CH)]],
                         dst, semA, add=True).wait()

    def fire_idx(i, cpos, xi, isem):
        gid = gbase + i * NCHUNK + cpos
        pltpu.async_copy(xTc.at[pl.ds(gid * IDXB, IDXB)], xi, isem)

    def drain_idx(xi, isem):
        pltpu.make_async_copy(xTc.at[pl.ds(0, IDXB)], xi, isem).wait()

    def fire_g(xi, gb, sem):
        for h in range(2):
            pltpu.async_copy(Wd.at[xi.at[pl.ds(h * HG, HG)]],
                             gb.at[pl.ds(h * HG, HG)], sem)

    def drain_g(gb, sem):
        for h in range(2):
            pltpu.make_async_copy(Wd.at[pl.ds(0, HG)],
                                  gb.at[pl.ds(h * HG, HG)], sem).wait()

    def drain_w():
        pltpu.make_async_copy(wrc, out.at[pl.ds(0, CH)], wsem).wait()

    def unpack_data(gb):
        def step(t, carry):
            for v in range(NV):
                for c in range(DM // LANES):
                    wrc[t, v, pl.ds(c * LANES, LANES)] = (
                        gb[v * CH + t, pl.ds(c * LANES, LANES)])
            return carry
        lax.fori_loop(0, CH, step, 0)

    def write(i, cpos):
        pltpu.async_copy(wrc, out.at[pl.ds(tbase + i * SEQ + cpos * CH, CH)],
                         wsem)

    pltpu.async_copy(wrc, out.at[pl.ds(tbase, CH)], wsem)

    for cpos in range(NCHUNK):
        drain_w()
        lo_all = (cpos + 1) * CH <= SEQ - 50
        hi_all = cpos * CH >= SEQ - 50

        def pos_step(t, carry):
            for c in range(2 * DM // LANES):
                wrc[t, 4 + c // 4, pl.ds((c % 4) * LANES, LANES)] = (
                    gb45[cpos, t, pl.ds(c * LANES, LANES)])
            for c in range(DM // LANES):
                if lo_all:
                    vec = w6lo[c]
                elif hi_all:
                    vec = w6hi[c]
                else:
                    vec = jnp.where(cpos * CH + t >= SEQ - 50,
                                    w6hi[c], w6lo[c])
                wrc[t, 6, pl.ds(c * LANES, LANES)] = vec
            return carry
        lax.fori_loop(0, CH, pos_step, 0)

        pltpu.async_copy(wrc, out.at[pl.ds(tbase + cpos * CH, CH)], wsem)
        fire_idx(jnp.int32(0), cpos, xidxA, isemA)
        fire_idx(jnp.int32(1), cpos, xidxB, isemB)
        drain_idx(xidxA, isemA)
        fire_g(xidxA, gbA, semA)
        drain_idx(xidxB, isemB)
        fire_g(xidxB, gbB, semB)

        def seg_body(j, carry):
            i = 2 * j
            drain_g(gbA, semA)
            fire_idx(jnp.minimum(i + 2, ROWS_PER_W - 1), cpos, xidxA, isemA)
            drain_w()
            unpack_data(gbA)
            write(i, cpos)
            drain_idx(xidxA, isemA)
            fire_g(xidxA, gbA, semA)

            drain_g(gbB, semB)
            fire_idx(jnp.minimum(i + 3, ROWS_PER_W - 1), cpos, xidxB, isemB)
            drain_w()
            unpack_data(gbB)
            write(i + 1, cpos)
            drain_idx(xidxB, isemB)
            fire_g(xidxB, gbB, semB)
            return carry
        lax.fori_loop(0, ROWS_PER_W // 2, seg_body, 0)

        drain_g(gbA, semA)
        drain_g(gbB, semB)

    drain_w()


def kernel(x, W0, W1, W2, W3, W4, W5, W6):
    offs = jnp.arange(NV, dtype=jnp.int32) * 100000
    xTc = jnp.transpose(
        x.astype(jnp.int32).reshape(BATCH, NCHUNK, CH, NV) + offs,
        (0, 1, 3, 2)).reshape(-1)
    Wd = jnp.concatenate(
        [jnp.concatenate([w, w], axis=1) for w in (W0, W1, W2, W3)], axis=0)
    z4, z5 = jnp.zeros_like(W4), jnp.zeros_like(W5)
    WL4 = jnp.concatenate([W4, z4], axis=1)
    WR5 = jnp.concatenate([z5, W5], axis=1)
    W6f = W6.reshape(2 * DM)

    mesh = plsc.VectorSubcoreMesh(core_axis_name="c", subcore_axis_name="s")
    f = pl.kernel(
        _body,
        out_type=jax.ShapeDtypeStruct((TOK, 7, DM), jnp.float32),
        mesh=mesh,
        scratch_types=[
            pltpu.VMEM((IDXB,), jnp.int32),
            pltpu.VMEM((IDXB,), jnp.int32),
            pltpu.VMEM((IPAD,), jnp.int32),
            pltpu.VMEM((IPAD,), jnp.int32),
            pltpu.VMEM((2 * DM,), jnp.float32),
            pltpu.VMEM((CH, 7, DM), jnp.float32),
            pltpu.VMEM((IDXB, 2 * DM), jnp.float32),
            pltpu.VMEM((IDXB, 2 * DM), jnp.float32),
            pltpu.VMEM((NCHUNK, CH, 2 * DM), jnp.float32),
            pltpu.SemaphoreType.DMA,
            pltpu.SemaphoreType.DMA,
            pltpu.SemaphoreType.DMA,
            pltpu.SemaphoreType.DMA,
            pltpu.SemaphoreType.DMA,
        ],
    )
    out = f(xTc, Wd, WL4, WR5, W6f)
    return out.reshape(BATCH, SEQ, 7, DM)
